# baseline (speedup 1.0000x reference)
import functools

import jax
import jax.numpy as jnp
from jax import lax
from jax.experimental import pallas as pl
from jax.experimental.pallas import tpu as pltpu
from jax.experimental.pallas import tpu_sc as plsc

_N = 50000
_NP = 50048
_E = 800000
_HID = 64
_HALF = 32
_NC = 2
_NS = 16
_NW = _NC * _NS
_ECH = 125
_ENCH = _E // (_NS * _ECH)
_EIB = 20
_NBUF = 5
_NROWS_T = _NP // _NS

_f32 = jnp.float32
_i32 = jnp.int32


def _sc_mesh():
    return plsc.VectorSubcoreMesh(core_axis_name="c", subcore_axis_name="s",
                                  num_cores=_NC, num_subcores=_NS)


def _sc_gather(table, idx3, d):
    nw, nch, ch = idx3.shape

    def body(table_hbm, idx_hbm, out_hbm, idxv, rows0, rows1, rows2,
             gs0, gs1, gs2, os0, os1, os2):
        c = lax.axis_index("c")
        s = lax.axis_index("s")
        w = c * _NS + s
        rows = (rows0, rows1, rows2)
        gsem = (gs0, gs1, gs2)
        osem = (os0, os1, os2)
        pltpu.sync_copy(idx_hbm.at[w], idxv)
        for k in range(min(3, nch)):
            pltpu.async_copy(table_hbm.at[idxv.at[k]], rows[k], gsem[k])
        for k in range(nch):
            b = k % 3
            pltpu.make_async_copy(table_hbm.at[idxv.at[k]], rows[b], gsem[b]).wait()
            pltpu.async_copy(rows[b], out_hbm.at[w, pl.ds(k * ch, ch)], osem[b])
            if k >= 1:
                bp = (k - 1) % 3
                pltpu.make_async_copy(rows[bp], out_hbm.at[w, pl.ds(0, ch)], osem[bp]).wait()
                kn = k + 2
                if kn < nch:
                    pltpu.async_copy(table_hbm.at[idxv.at[kn]], rows[bp], gsem[bp])
        bl = (nch - 1) % 3
        pltpu.make_async_copy(rows[bl], out_hbm.at[w, pl.ds(0, ch)], osem[bl]).wait()

    f = pl.kernel(
        body,
        out_type=jax.ShapeDtypeStruct((nw, nch * ch, d), _f32),
        mesh=_sc_mesh(),
        compiler_params=pltpu.CompilerParams(use_tc_tiling_on_sc=False),
        scratch_types=[pltpu.VMEM((nch, ch), _i32)]
        + [pltpu.VMEM((ch, d), _f32)] * 3
        + [pltpu.SemaphoreType.DMA] * 6,
    )
    return f(table, idx3).reshape(nw * nch * ch, d)


def _sc_segsum(hsplit, src2, dst3, zblk):
    def body(h_hbm, src_hbm, dst_hbm, z_hbm, out_hbm, srcv0, dstv0, srcv1, dstv1,
             rows0, rows1, rows2, rows3, rows4, acc,
             gs0, gs1, gs2, gs3, gs4, ss0, ss1, ss2, ss3, ss4, is0, is1):
        c = lax.axis_index("c")
        s = lax.axis_index("s")
        r0 = s * _NROWS_T
        rows = (rows0, rows1, rows2, rows3, rows4)
        gsem = (gs0, gs1, gs2, gs3, gs4)
        ssem = (ss0, ss1, ss2, ss3, ss4)
        slots = ((srcv0, dstv0, is0), (srcv1, dstv1, is1))
        nblk = _ENCH // _EIB
        pltpu.sync_copy(z_hbm, acc.at[pl.ds(r0, _NROWS_T)])
        pltpu.async_copy(src_hbm.at[c, s, pl.ds(0, _EIB)], srcv0, is0)
        pltpu.async_copy(dst_hbm.at[s, pl.ds(0, _EIB)], dstv0, is0)
        plsc.subcore_barrier()

        def run_block(j, sv, dv, isem, sv_n, dv_n, isem_n):
            pltpu.make_async_copy(src_hbm.at[c, s, pl.ds(0, _EIB)], sv, isem).wait()
            pltpu.make_async_copy(dst_hbm.at[s, pl.ds(0, _EIB)], dv, isem).wait()
            jn = jnp.minimum(j + 1, nblk - 1) * _EIB
            pltpu.async_copy(src_hbm.at[c, s, pl.ds(jn, _EIB)], sv_n, isem_n)
            pltpu.async_copy(dst_hbm.at[s, pl.ds(jn, _EIB)], dv_n, isem_n)
            for k in range(_NBUF):
                pltpu.async_copy(h_hbm.at[sv.at[k]], rows[k], gsem[k])
            for k in range(_EIB):
                b = k % _NBUF
                pltpu.make_async_copy(h_hbm.at[sv.at[k]], rows[b], gsem[b]).wait()
                pltpu.async_copy(rows[b], acc.at[dv.at[k]], ssem[b], add=True)
                if k >= 2:
                    bp = (k - 2) % _NBUF
                    pltpu.make_async_copy(rows[bp], acc.at[dv.at[k]], ssem[bp]).wait()
                    kn = k + _NBUF - 2
                    if kn < _EIB:
                        pltpu.async_copy(h_hbm.at[sv.at[kn]], rows[bp], gsem[bp])
            for k in (_EIB - 2, _EIB - 1):
                bl = k % _NBUF
                pltpu.make_async_copy(rows[bl], acc.at[dv.at[0]], ssem[bl]).wait()

        def sblk(i, carry):
            run_block(2 * i, *slots[0], *slots[1])
            run_block(2 * i + 1, *slots[1], *slots[0])
            return carry

        lax.fori_loop(0, nblk // 2, sblk, 0)
        pltpu.make_async_copy(src_hbm.at[c, s, pl.ds(0, _EIB)], srcv0, is0).wait()
        pltpu.make_async_copy(dst_hbm.at[s, pl.ds(0, _EIB)], dstv0, is0).wait()
        plsc.subcore_barrier()
        pltpu.sync_copy(acc.at[pl.ds(r0, _NROWS_T)], out_hbm.at[c, pl.ds(r0, _NROWS_T)])

    f = pl.kernel(
        body,
        out_type=jax.ShapeDtypeStruct((_NC, _NP, _HALF), _f32),
        mesh=_sc_mesh(),
        compiler_params=pltpu.CompilerParams(use_tc_tiling_on_sc=False),
        scratch_types=[pltpu.VMEM((_EIB, _ECH), _i32)] * 4
        + [pltpu.VMEM((_ECH, _HALF), _f32)] * _NBUF + [
            pltpu.VMEM_SHARED((_NP, _HALF), _f32),
        ] + [pltpu.SemaphoreType.DMA] * (2 * _NBUF + 2),
    )
    return f(hsplit, src2, dst3, zblk)


def _sc_degree(dst3, ones_blk, zblk):
    def body(ones_hbm, dst_hbm, z_hbm, out_hbm, dstv, rows, acc, ssem):
        c = lax.axis_index("c")
        s = lax.axis_index("s")
        r0 = s * _NROWS_T
        pltpu.sync_copy(z_hbm, acc.at[pl.ds(r0, _NROWS_T)])
        pltpu.sync_copy(ones_hbm, rows)
        plsc.subcore_barrier()

        def blk(j2, carry):
            pltpu.sync_copy(dst_hbm.at[s, pl.ds(j2 * _EIB, _EIB)], dstv)
            for j in range(_EIB):
                pltpu.async_copy(rows, acc.at[dstv.at[j]], ssem, add=True)
            for j in range(_EIB):
                pltpu.make_async_copy(rows, acc.at[dstv.at[j]], ssem).wait()
            return carry

        lax.fori_loop(0, _ENCH // _EIB, blk, 0)
        plsc.subcore_barrier()

        @pl.when(c == 0)
        def _():
            pltpu.sync_copy(acc.at[pl.ds(r0, _NROWS_T)], out_hbm.at[pl.ds(r0, _NROWS_T)])

    f = pl.kernel(
        body,
        out_type=jax.ShapeDtypeStruct((_NP, _HALF), _f32),
        mesh=_sc_mesh(),
        compiler_params=pltpu.CompilerParams(use_tc_tiling_on_sc=False),
        scratch_types=[
            pltpu.VMEM((_EIB, _ECH), _i32),
            pltpu.VMEM((_ECH, _HALF), _f32),
            pltpu.VMEM_SHARED((_NP, _HALF), _f32),
            pltpu.SemaphoreType.DMA,
        ],
    )
    return f(ones_blk, dst3, zblk)


_PB = _NROWS_T


def _tc_proj(rows_pad, w_t, b):
    def body(x_ref, w_ref, b_ref, o_ref):
        x = jnp.dot(x_ref[...], w_ref[...], preferred_element_type=_f32) + b_ref[...]
        o_ref[0] = x[:, :_HALF]
        o_ref[1] = x[:, _HALF:]

    return pl.pallas_call(
        body,
        grid=(_NP // _PB,),
        in_specs=[
            pl.BlockSpec((_PB, rows_pad.shape[1]), lambda b: (b, 0)),
            pl.BlockSpec(w_t.shape, lambda b: (0, 0)),
            pl.BlockSpec((1, _HID), lambda b: (0, 0)),
        ],
        out_specs=pl.BlockSpec((_NC, _PB, _HALF), lambda b: (0, b, 0)),
        out_shape=jax.ShapeDtypeStruct((_NC, _NP, _HALF), _f32),
    )(rows_pad, w_t, b.reshape(1, _HID))


def _tc_gru(s2, h2, deg, wm_t, bm, wi_t, bi, wh_t, bh):
    def body(s_ref, h_ref, d_ref, wm_ref, bm_ref,
             wir_ref, wiz_ref, win_ref, bir_ref, biz_ref, bin_ref,
             whr_ref, whz_ref, whn_ref, bhr_ref, bhz_ref, bhn_ref, o_ref):
        sv = jnp.concatenate([s_ref[0], s_ref[1]], axis=-1)
        hv = jnp.concatenate([h_ref[0], h_ref[1]], axis=-1)
        x = (jnp.dot(sv, wm_ref[...], preferred_element_type=_f32)
             + d_ref[...] * bm_ref[...])
        i_r = jnp.dot(x, wir_ref[...], preferred_element_type=_f32) + bir_ref[...]
        i_z = jnp.dot(x, wiz_ref[...], preferred_element_type=_f32) + biz_ref[...]
        i_n = jnp.dot(x, win_ref[...], preferred_element_type=_f32) + bin_ref[...]
        h_r = jnp.dot(hv, whr_ref[...], preferred_element_type=_f32) + bhr_ref[...]
        h_z = jnp.dot(hv, whz_ref[...], preferred_element_type=_f32) + bhz_ref[...]
        h_n = jnp.dot(hv, whn_ref[...], preferred_element_type=_f32) + bhn_ref[...]
        r = jax.nn.sigmoid(i_r + h_r)
        z = jax.nn.sigmoid(i_z + h_z)
        n = jnp.tanh(i_n + r * h_n)
        hn = (1.0 - z) * n + z * hv
        o_ref[0] = hn[:, :_HALF]
        o_ref[1] = hn[:, _HALF:]

    full = lambda a: pl.BlockSpec(a.shape, lambda b: (0, 0))
    args = (s2, h2, deg, wm_t, bm, wi_t[0], wi_t[1], wi_t[2], bi[0], bi[1], bi[2],
            wh_t[0], wh_t[1], wh_t[2], bh[0], bh[1], bh[2])
    split_spec = pl.BlockSpec((_NC, _PB, _HALF), lambda b: (0, b, 0))
    in_specs = [split_spec, split_spec, pl.BlockSpec((_PB, 1), lambda b: (b, 0))]
    in_specs += [full(a) for a in args[3:]]
    return pl.pallas_call(
        body,
        grid=(_NP // _PB,),
        in_specs=in_specs,
        out_specs=split_spec,
        out_shape=jax.ShapeDtypeStruct((_NC, _NP, _HALF), _f32),
    )(*args)


def _tc_classifier(pg0, pg1, y, w1_t, b1, w2_t, b2):
    p = y.shape[0]

    def body(a_ref, b_ref, c_ref, d_ref, y_ref, w1_ref, b1_ref, w2_ref, b2_ref,
             lg_ref, ls_ref):
        h = jnp.concatenate([a_ref[...], b_ref[...], c_ref[...], d_ref[...]],
                            axis=-1)
        h1 = jnp.dot(h, w1_ref[...], preferred_element_type=_f32) + b1_ref[...]
        h1 = jnp.maximum(h1, 0.0)
        u = jnp.dot(h1, w2_ref[...], preferred_element_type=_f32) + b2_ref[...]
        prob = jax.nn.sigmoid(u)
        lg_ref[...] = prob
        yv = y_ref[...]
        ll = (yv * jnp.maximum(jnp.log(prob), -100.0)
              + (1.0 - yv) * jnp.maximum(jnp.log(1.0 - prob), -100.0))
        ls_ref[...] = jnp.broadcast_to(-jnp.mean(ll), (1, 1))

    half_block = lambda i: pl.BlockSpec((p, _HALF), lambda b: (i, 0))
    full = lambda a: pl.BlockSpec(a.shape, lambda b: (0, 0))
    return pl.pallas_call(
        body,
        grid=(1,),
        in_specs=[half_block(0), half_block(1), half_block(0), half_block(1),
                  full(y), full(w1_t), full(b1), full(w2_t), full(b2)],
        out_specs=[pl.BlockSpec((p, 1), lambda b: (0, 0)),
                   pl.BlockSpec((1, 1), lambda b: (0, 0))],
        out_shape=[jax.ShapeDtypeStruct((p, 1), _f32),
                   jax.ShapeDtypeStruct((1, 1), _f32)],
    )(pg0, pg0, pg1, pg1, y, w1_t, b1, w2_t, b2)


def kernel(emb_table, proj_W, proj_b, msg_W_0, msg_b_0, gru_Wih_0, gru_Whh_0,
           gru_bih_0, gru_bhh_0, msg_W_1, msg_b_1, gru_Wih_1, gru_Whh_1,
           gru_bih_1, gru_bhh_1, cla1_W, cla1_b, cla2_W, cla2_b,
           emb_ind_0, emb_ind_1, adj_0, adj_1, prop_ind_0, prop_ind_1, labels):
    zblk = jnp.zeros((_NROWS_T, _HALF), _f32)
    ones_blk = jnp.ones((_ECH, _HALF), _f32)

    layers = []
    for msg_W, msg_b, Wih, Whh, bih, bhh in (
            (msg_W_0, msg_b_0, gru_Wih_0, gru_Whh_0, gru_bih_0, gru_bhh_0),
            (msg_W_1, msg_b_1, gru_Wih_1, gru_Whh_1, gru_bih_1, gru_bhh_1)):
        wi_t = tuple(Wih[k * _HID:(k + 1) * _HID].T for k in range(3))
        wh_t = tuple(Whh[k * _HID:(k + 1) * _HID].T for k in range(3))
        bi = tuple(bih[k * _HID:(k + 1) * _HID].reshape(1, _HID) for k in range(3))
        bh = tuple(bhh[k * _HID:(k + 1) * _HID].reshape(1, _HID) for k in range(3))
        layers.append((msg_W.T, msg_b.reshape(1, _HID), wi_t, bi, wh_t, bh))

    emb_nch, emb_ch = 14, 112
    emb_pad = _NW * emb_nch * emb_ch - _N

    hs, degs, src2s, dst3s = [], [], [], []
    for emb_ind, adj in ((emb_ind_0, adj_0), (emb_ind_1, adj_1)):
        ei = jnp.concatenate([emb_ind.astype(_i32), jnp.zeros((emb_pad,), _i32)])
        rows = _sc_gather(emb_table, ei.reshape(_NW, emb_nch, emb_ch), 128)
        h2 = _tc_proj(rows, proj_W.T, proj_b)
        src = adj[:, 0].astype(_i32).reshape(_NS, _ENCH, _ECH)
        dst3 = adj[:, 1].astype(_i32).reshape(_NS, _ENCH, _ECH)
        src2 = jnp.stack([src, src + _NP])
        deg = _sc_degree(dst3, ones_blk, zblk)[:, :1]
        hs.append(h2)
        degs.append(deg)
        src2s.append(src2)
        dst3s.append(dst3)

    for wm_t, bm, wi_t, bi, wh_t, bh in layers:
        for _ in range(3):
            s2 = [_sc_segsum(hs[g].reshape(_NC * _NP, _HALF), src2s[g], dst3s[g],
                             zblk) for g in range(2)]
            hs = [_tc_gru(s2[g], hs[g], degs[g], wm_t, bm, wi_t, bi, wh_t, bh)
                  for g in range(2)]

    pgs = []
    for g, prop_ind in enumerate((prop_ind_0, prop_ind_1)):
        pi = prop_ind.astype(_i32)
        pidx = jnp.concatenate([pi, pi + _NP]).reshape(_NW, 2, 128)
        pgs.append(_sc_gather(hs[g].reshape(_NC * _NP, _HALF), pidx, _HALF))

    y = labels.astype(_f32).reshape(-1, 1)
    logits, loss = _tc_classifier(pgs[0], pgs[1], y, cla1_W.T,
                                  cla1_b.reshape(1, _HID), cla2_W.T,
                                  cla2_b.reshape(1, 1))
    return logits, loss.reshape(())


# speedup vs baseline: 1.0445x; 1.0445x over previous
import functools

import jax
import jax.numpy as jnp
from jax import lax
from jax.experimental import pallas as pl
from jax.experimental.pallas import tpu as pltpu
from jax.experimental.pallas import tpu_sc as plsc

_N = 50000
_NP = 50048
_E = 800000
_HID = 64
_HALF = 32
_NC = 2
_NS = 16
_NW = _NC * _NS
_ECH = 125
_ENCH = _E // (_NS * _ECH)
_EIB = 20
_NBUF = 5
_NROWS_T = _NP // _NS

_f32 = jnp.float32
_i32 = jnp.int32


def _sc_mesh():
    return plsc.VectorSubcoreMesh(core_axis_name="c", subcore_axis_name="s",
                                  num_cores=_NC, num_subcores=_NS)


def _sc_gather(table, idx3, d):
    nw, nch, ch = idx3.shape

    def body(table_hbm, idx_hbm, out_hbm, idxv, rows0, rows1, rows2,
             gs0, gs1, gs2, os0, os1, os2):
        c = lax.axis_index("c")
        s = lax.axis_index("s")
        w = c * _NS + s
        rows = (rows0, rows1, rows2)
        gsem = (gs0, gs1, gs2)
        osem = (os0, os1, os2)
        pltpu.sync_copy(idx_hbm.at[w], idxv)
        for k in range(min(3, nch)):
            pltpu.async_copy(table_hbm.at[idxv.at[k]], rows[k], gsem[k])
        for k in range(nch):
            b = k % 3
            pltpu.make_async_copy(table_hbm.at[idxv.at[k]], rows[b], gsem[b]).wait()
            pltpu.async_copy(rows[b], out_hbm.at[w, pl.ds(k * ch, ch)], osem[b])
            if k >= 1:
                bp = (k - 1) % 3
                pltpu.make_async_copy(rows[bp], out_hbm.at[w, pl.ds(0, ch)], osem[bp]).wait()
                kn = k + 2
                if kn < nch:
                    pltpu.async_copy(table_hbm.at[idxv.at[kn]], rows[bp], gsem[bp])
        bl = (nch - 1) % 3
        pltpu.make_async_copy(rows[bl], out_hbm.at[w, pl.ds(0, ch)], osem[bl]).wait()

    f = pl.kernel(
        body,
        out_type=jax.ShapeDtypeStruct((nw, nch * ch, d), _f32),
        mesh=_sc_mesh(),
        compiler_params=pltpu.CompilerParams(use_tc_tiling_on_sc=False),
        scratch_types=[pltpu.VMEM((nch, ch), _i32)]
        + [pltpu.VMEM((ch, d), _f32)] * 3
        + [pltpu.SemaphoreType.DMA] * 6,
    )
    return f(table, idx3).reshape(nw * nch * ch, d)


def _sc_segsum(hsplit, src2, dst3, zblk):
    def body(h_hbm, src_hbm, dst_hbm, z_hbm, out_hbm, srcv0, dstv0, srcv1, dstv1,
             rows0, rows1, rows2, rows3, rows4, acc,
             gs0, gs1, gs2, gs3, gs4, ss0, ss1, ss2, ss3, ss4, is0, is1):
        c = lax.axis_index("c")
        s = lax.axis_index("s")
        r0 = s * _NROWS_T
        rows = (rows0, rows1, rows2, rows3, rows4)
        gsem = (gs0, gs1, gs2, gs3, gs4)
        ssem = (ss0, ss1, ss2, ss3, ss4)
        slots = ((srcv0, dstv0, is0), (srcv1, dstv1, is1))
        nblk = _ENCH // _EIB
        pltpu.async_copy(z_hbm, acc.at[pl.ds(r0, _NROWS_T)], ss0)
        pltpu.async_copy(src_hbm.at[c, s, pl.ds(0, _EIB)], srcv0, is0)
        pltpu.async_copy(dst_hbm.at[s, pl.ds(0, _EIB)], dstv0, is0)
        pltpu.make_async_copy(z_hbm, acc.at[pl.ds(r0, _NROWS_T)], ss0).wait()
        plsc.subcore_barrier()

        def run_block(j, sv, dv, isem, sv_n, dv_n, isem_n):
            pltpu.make_async_copy(src_hbm.at[c, s, pl.ds(0, _EIB)], sv, isem).wait()
            pltpu.make_async_copy(dst_hbm.at[s, pl.ds(0, _EIB)], dv, isem).wait()
            jn = jnp.minimum(j + 1, nblk - 1) * _EIB
            pltpu.async_copy(src_hbm.at[c, s, pl.ds(jn, _EIB)], sv_n, isem_n)
            pltpu.async_copy(dst_hbm.at[s, pl.ds(jn, _EIB)], dv_n, isem_n)
            for k in range(_NBUF):
                pltpu.async_copy(h_hbm.at[sv.at[k]], rows[k], gsem[k])
            for k in range(_EIB):
                b = k % _NBUF
                pltpu.make_async_copy(h_hbm.at[sv.at[k]], rows[b], gsem[b]).wait()
                pltpu.async_copy(rows[b], acc.at[dv.at[k]], ssem[b], add=True)
                if k >= 1:
                    bp = (k - 1) % _NBUF
                    pltpu.make_async_copy(rows[bp], acc.at[dv.at[k]], ssem[bp]).wait()
                    kn = k + _NBUF - 1
                    if kn < _EIB:
                        pltpu.async_copy(h_hbm.at[sv.at[kn]], rows[bp], gsem[bp])
            bl = (_EIB - 1) % _NBUF
            pltpu.make_async_copy(rows[bl], acc.at[dv.at[0]], ssem[bl]).wait()

        def sblk(i, carry):
            run_block(2 * i, *slots[0], *slots[1])
            run_block(2 * i + 1, *slots[1], *slots[0])
            return carry

        lax.fori_loop(0, nblk // 2, sblk, 0)
        pltpu.make_async_copy(src_hbm.at[c, s, pl.ds(0, _EIB)], srcv0, is0).wait()
        pltpu.make_async_copy(dst_hbm.at[s, pl.ds(0, _EIB)], dstv0, is0).wait()
        plsc.subcore_barrier()
        pltpu.sync_copy(acc.at[pl.ds(r0, _NROWS_T)], out_hbm.at[c, pl.ds(r0, _NROWS_T)])

    f = pl.kernel(
        body,
        out_type=jax.ShapeDtypeStruct((_NC, _NP, _HALF), _f32),
        mesh=_sc_mesh(),
        compiler_params=pltpu.CompilerParams(use_tc_tiling_on_sc=False),
        scratch_types=[pltpu.VMEM((_EIB, _ECH), _i32)] * 4
        + [pltpu.VMEM((_ECH, _HALF), _f32)] * _NBUF + [
            pltpu.VMEM_SHARED((_NP, _HALF), _f32),
        ] + [pltpu.SemaphoreType.DMA] * (2 * _NBUF + 2),
    )
    return f(hsplit, src2, dst3, zblk)


def _sc_degree(dst3, ones_blk, zblk):
    def body(ones_hbm, dst_hbm, z_hbm, out_hbm, dstv, rows, acc, ssem):
        c = lax.axis_index("c")
        s = lax.axis_index("s")
        r0 = s * _NROWS_T
        pltpu.sync_copy(z_hbm, acc.at[pl.ds(r0, _NROWS_T)])
        pltpu.sync_copy(ones_hbm, rows)
        plsc.subcore_barrier()

        def blk(j2, carry):
            pltpu.sync_copy(dst_hbm.at[s, pl.ds(j2 * _EIB, _EIB)], dstv)
            for j in range(_EIB):
                pltpu.async_copy(rows, acc.at[dstv.at[j]], ssem, add=True)
            for j in range(_EIB):
                pltpu.make_async_copy(rows, acc.at[dstv.at[j]], ssem).wait()
            return carry

        lax.fori_loop(0, _ENCH // _EIB, blk, 0)
        plsc.subcore_barrier()

        @pl.when(c == 0)
        def _():
            pltpu.sync_copy(acc.at[pl.ds(r0, _NROWS_T)], out_hbm.at[pl.ds(r0, _NROWS_T)])

    f = pl.kernel(
        body,
        out_type=jax.ShapeDtypeStruct((_NP, _HALF), _f32),
        mesh=_sc_mesh(),
        compiler_params=pltpu.CompilerParams(use_tc_tiling_on_sc=False),
        scratch_types=[
            pltpu.VMEM((_EIB, _ECH), _i32),
            pltpu.VMEM((_ECH, _HALF), _f32),
            pltpu.VMEM_SHARED((_NP, _HALF), _f32),
            pltpu.SemaphoreType.DMA,
        ],
    )
    return f(ones_blk, dst3, zblk)


_PB = _NROWS_T


def _tc_proj(rows_pad, w_t, b):
    def body(x_ref, w_ref, b_ref, o_ref):
        x = jnp.dot(x_ref[...], w_ref[...], preferred_element_type=_f32) + b_ref[...]
        o_ref[0] = x[:, :_HALF]
        o_ref[1] = x[:, _HALF:]

    return pl.pallas_call(
        body,
        grid=(_NP // _PB,),
        in_specs=[
            pl.BlockSpec((_PB, rows_pad.shape[1]), lambda b: (b, 0)),
            pl.BlockSpec(w_t.shape, lambda b: (0, 0)),
            pl.BlockSpec((1, _HID), lambda b: (0, 0)),
        ],
        out_specs=pl.BlockSpec((_NC, _PB, _HALF), lambda b: (0, b, 0)),
        out_shape=jax.ShapeDtypeStruct((_NC, _NP, _HALF), _f32),
    )(rows_pad, w_t, b.reshape(1, _HID))


def _tc_gru(s2, h2, deg, wm_t, bm, wih_t, bih, whh_t, bhh):
    def body(s_ref, h_ref, d_ref, wm_ref, bm_ref, wih_ref, bih_ref,
             whh_ref, bhh_ref, o_ref):
        sv = jnp.concatenate([s_ref[0], s_ref[1]], axis=-1)
        hv = jnp.concatenate([h_ref[0], h_ref[1]], axis=-1)
        x = (jnp.dot(sv, wm_ref[...], preferred_element_type=_f32)
             + d_ref[...] * bm_ref[...])
        gi = jnp.dot(x, wih_ref[...], preferred_element_type=_f32) + bih_ref[...]
        gh = jnp.dot(hv, whh_ref[...], preferred_element_type=_f32) + bhh_ref[...]
        r = jax.nn.sigmoid(gi[:, :_HID] + gh[:, :_HID])
        z = jax.nn.sigmoid(gi[:, _HID:2 * _HID] + gh[:, _HID:2 * _HID])
        n = jnp.tanh(gi[:, 2 * _HID:] + r * gh[:, 2 * _HID:])
        hn = (1.0 - z) * n + z * hv
        o_ref[0] = hn[:, :_HALF]
        o_ref[1] = hn[:, _HALF:]

    full = lambda a: pl.BlockSpec(a.shape, lambda b: (0, 0))
    args = (s2, h2, deg, wm_t, bm, wih_t, bih, whh_t, bhh)
    split_spec = pl.BlockSpec((_NC, _PB, _HALF), lambda b: (0, b, 0))
    in_specs = [split_spec, split_spec, pl.BlockSpec((_PB, 1), lambda b: (b, 0))]
    in_specs += [full(a) for a in args[3:]]
    return pl.pallas_call(
        body,
        grid=(_NP // _PB,),
        in_specs=in_specs,
        out_specs=split_spec,
        out_shape=jax.ShapeDtypeStruct((_NC, _NP, _HALF), _f32),
    )(*args)


def _tc_classifier(pg0, pg1, y, w1_t, b1, w2_t, b2):
    p = y.shape[0]

    def body(a_ref, b_ref, c_ref, d_ref, y_ref, w1_ref, b1_ref, w2_ref, b2_ref,
             lg_ref, ls_ref):
        h = jnp.concatenate([a_ref[...], b_ref[...], c_ref[...], d_ref[...]],
                            axis=-1)
        h1 = jnp.dot(h, w1_ref[...], preferred_element_type=_f32) + b1_ref[...]
        h1 = jnp.maximum(h1, 0.0)
        u = jnp.dot(h1, w2_ref[...], preferred_element_type=_f32) + b2_ref[...]
        prob = jax.nn.sigmoid(u)
        lg_ref[...] = prob
        yv = y_ref[...]
        ll = (yv * jnp.maximum(jnp.log(prob), -100.0)
              + (1.0 - yv) * jnp.maximum(jnp.log(1.0 - prob), -100.0))
        ls_ref[...] = jnp.broadcast_to(-jnp.mean(ll), (1, 1))

    half_block = lambda i: pl.BlockSpec((p, _HALF), lambda b: (i, 0))
    full = lambda a: pl.BlockSpec(a.shape, lambda b: (0, 0))
    return pl.pallas_call(
        body,
        grid=(1,),
        in_specs=[half_block(0), half_block(1), half_block(0), half_block(1),
                  full(y), full(w1_t), full(b1), full(w2_t), full(b2)],
        out_specs=[pl.BlockSpec((p, 1), lambda b: (0, 0)),
                   pl.BlockSpec((1, 1), lambda b: (0, 0))],
        out_shape=[jax.ShapeDtypeStruct((p, 1), _f32),
                   jax.ShapeDtypeStruct((1, 1), _f32)],
    )(pg0, pg0, pg1, pg1, y, w1_t, b1, w2_t, b2)


def kernel(emb_table, proj_W, proj_b, msg_W_0, msg_b_0, gru_Wih_0, gru_Whh_0,
           gru_bih_0, gru_bhh_0, msg_W_1, msg_b_1, gru_Wih_1, gru_Whh_1,
           gru_bih_1, gru_bhh_1, cla1_W, cla1_b, cla2_W, cla2_b,
           emb_ind_0, emb_ind_1, adj_0, adj_1, prop_ind_0, prop_ind_1, labels):
    zblk = jnp.zeros((_NROWS_T, _HALF), _f32)
    ones_blk = jnp.ones((_ECH, _HALF), _f32)

    layers = []
    for msg_W, msg_b, Wih, Whh, bih, bhh in (
            (msg_W_0, msg_b_0, gru_Wih_0, gru_Whh_0, gru_bih_0, gru_bhh_0),
            (msg_W_1, msg_b_1, gru_Wih_1, gru_Whh_1, gru_bih_1, gru_bhh_1)):
        layers.append((msg_W.T, msg_b.reshape(1, _HID), Wih.T,
                       bih.reshape(1, 3 * _HID), Whh.T, bhh.reshape(1, 3 * _HID)))

    emb_nch, emb_ch = 14, 112
    emb_pad = _NW * emb_nch * emb_ch - _N

    hs, degs, src2s, dst3s = [], [], [], []
    for emb_ind, adj in ((emb_ind_0, adj_0), (emb_ind_1, adj_1)):
        ei = jnp.concatenate([emb_ind.astype(_i32), jnp.zeros((emb_pad,), _i32)])
        rows = _sc_gather(emb_table, ei.reshape(_NW, emb_nch, emb_ch), 128)
        h2 = _tc_proj(rows, proj_W.T, proj_b)
        src = adj[:, 0].astype(_i32).reshape(_NS, _ENCH, _ECH)
        dst3 = adj[:, 1].astype(_i32).reshape(_NS, _ENCH, _ECH)
        src2 = jnp.stack([src, src + _NP])
        deg = _sc_degree(dst3, ones_blk, zblk)[:, :1]
        hs.append(h2)
        degs.append(deg)
        src2s.append(src2)
        dst3s.append(dst3)

    for wm_t, bm, wih_t, bih, whh_t, bhh in layers:
        for _ in range(3):
            s2 = [_sc_segsum(hs[g].reshape(_NC * _NP, _HALF), src2s[g], dst3s[g],
                             zblk) for g in range(2)]
            hs = [_tc_gru(s2[g], hs[g], degs[g], wm_t, bm, wih_t, bih,
                          whh_t, bhh) for g in range(2)]

    pgs = []
    for g, prop_ind in enumerate((prop_ind_0, prop_ind_1)):
        pi = prop_ind.astype(_i32)
        pidx = jnp.concatenate([pi, pi + _NP]).reshape(_NW, 2, 128)
        pgs.append(_sc_gather(hs[g].reshape(_NC * _NP, _HALF), pidx, _HALF))

    y = labels.astype(_f32).reshape(-1, 1)
    logits, loss = _tc_classifier(pgs[0], pgs[1], y, cla1_W.T,
                                  cla1_b.reshape(1, _HID), cla2_W.T,
                                  cla2_b.reshape(1, 1))
    return logits, loss.reshape(())


# speedup vs baseline: 1.0615x; 1.0162x over previous
import functools

import jax
import jax.numpy as jnp
from jax import lax
from jax.experimental import pallas as pl
from jax.experimental.pallas import tpu as pltpu
from jax.experimental.pallas import tpu_sc as plsc

_N = 50000
_NP = 50048
_E = 800000
_HID = 64
_HALF = 32
_NC = 2
_NS = 16
_NW = _NC * _NS
_ECH = 125
_ENCH = _E // (_NS * _ECH)
_EIB = 20
_NBUF = 5
_NROWS_T = _NP // _NS

_f32 = jnp.float32
_i32 = jnp.int32


def _sc_mesh():
    return plsc.VectorSubcoreMesh(core_axis_name="c", subcore_axis_name="s",
                                  num_cores=_NC, num_subcores=_NS)


def _sc_gather(table, idx3, d):
    nw, nch, ch = idx3.shape

    def body(table_hbm, idx_hbm, out_hbm, idxv, rows0, rows1, rows2,
             gs0, gs1, gs2, os0, os1, os2):
        c = lax.axis_index("c")
        s = lax.axis_index("s")
        w = c * _NS + s
        rows = (rows0, rows1, rows2)
        gsem = (gs0, gs1, gs2)
        osem = (os0, os1, os2)
        pltpu.sync_copy(idx_hbm.at[w], idxv)
        for k in range(min(3, nch)):
            pltpu.async_copy(table_hbm.at[idxv.at[k]], rows[k], gsem[k])
        for k in range(nch):
            b = k % 3
            pltpu.make_async_copy(table_hbm.at[idxv.at[k]], rows[b], gsem[b]).wait()
            pltpu.async_copy(rows[b], out_hbm.at[w, pl.ds(k * ch, ch)], osem[b])
            if k >= 1:
                bp = (k - 1) % 3
                pltpu.make_async_copy(rows[bp], out_hbm.at[w, pl.ds(0, ch)], osem[bp]).wait()
                kn = k + 2
                if kn < nch:
                    pltpu.async_copy(table_hbm.at[idxv.at[kn]], rows[bp], gsem[bp])
        bl = (nch - 1) % 3
        pltpu.make_async_copy(rows[bl], out_hbm.at[w, pl.ds(0, ch)], osem[bl]).wait()

    f = pl.kernel(
        body,
        out_type=jax.ShapeDtypeStruct((nw, nch * ch, d), _f32),
        mesh=_sc_mesh(),
        compiler_params=pltpu.CompilerParams(use_tc_tiling_on_sc=False),
        scratch_types=[pltpu.VMEM((nch, ch), _i32)]
        + [pltpu.VMEM((ch, d), _f32)] * 3
        + [pltpu.SemaphoreType.DMA] * 6,
    )
    return f(table, idx3).reshape(nw * nch * ch, d)


def _sc_segsum(hsplit, src2, dst3, zblk):
    def body(h_hbm, src_hbm, dst_hbm, z_hbm, out_hbm, srcv0, dstv0, srcv1, dstv1,
             rows0, rows1, rows2, rows3, rows4, acc,
             gs0, gs1, gs2, gs3, gs4, ss0, ss1, ss2, ss3, ss4, is0, is1):
        c = lax.axis_index("c")
        s = lax.axis_index("s")
        r0 = s * _NROWS_T
        rows = (rows0, rows1, rows2, rows3, rows4)
        gsem = (gs0, gs1, gs2, gs3, gs4)
        ssem = (ss0, ss1, ss2, ss3, ss4)
        slots = ((srcv0, dstv0, is0), (srcv1, dstv1, is1))
        nblk = _ENCH // _EIB
        pltpu.async_copy(z_hbm, acc.at[pl.ds(r0, _NROWS_T)], ss0)
        pltpu.async_copy(src_hbm.at[c, s, pl.ds(0, _EIB)], srcv0, is0)
        pltpu.async_copy(dst_hbm.at[s, pl.ds(0, _EIB)], dstv0, is0)
        pltpu.make_async_copy(z_hbm, acc.at[pl.ds(r0, _NROWS_T)], ss0).wait()
        plsc.subcore_barrier()

        def run_block(j, sv, dv, isem, sv_n, dv_n, isem_n):
            pltpu.make_async_copy(src_hbm.at[c, s, pl.ds(0, _EIB)], sv, isem).wait()
            pltpu.make_async_copy(dst_hbm.at[s, pl.ds(0, _EIB)], dv, isem).wait()
            jn = jnp.minimum(j + 1, nblk - 1) * _EIB
            pltpu.async_copy(src_hbm.at[c, s, pl.ds(jn, _EIB)], sv_n, isem_n)
            pltpu.async_copy(dst_hbm.at[s, pl.ds(jn, _EIB)], dv_n, isem_n)
            for k in range(_NBUF):
                pltpu.async_copy(h_hbm.at[sv.at[k]], rows[k], gsem[k])
            for k in range(_EIB):
                b = k % _NBUF
                pltpu.make_async_copy(h_hbm.at[sv.at[k]], rows[b], gsem[b]).wait()
                pltpu.async_copy(rows[b], acc.at[dv.at[k]], ssem[b], add=True)
                if k >= 1:
                    bp = (k - 1) % _NBUF
                    pltpu.make_async_copy(rows[bp], acc.at[dv.at[k]], ssem[bp]).wait()
                    kn = k + _NBUF - 1
                    if kn < _EIB:
                        pltpu.async_copy(h_hbm.at[sv.at[kn]], rows[bp], gsem[bp])
            bl = (_EIB - 1) % _NBUF
            pltpu.make_async_copy(rows[bl], acc.at[dv.at[0]], ssem[bl]).wait()

        def sblk(i, carry):
            run_block(2 * i, *slots[0], *slots[1])
            run_block(2 * i + 1, *slots[1], *slots[0])
            return carry

        lax.fori_loop(0, nblk // 2, sblk, 0)
        pltpu.make_async_copy(src_hbm.at[c, s, pl.ds(0, _EIB)], srcv0, is0).wait()
        pltpu.make_async_copy(dst_hbm.at[s, pl.ds(0, _EIB)], dstv0, is0).wait()
        plsc.subcore_barrier()
        pltpu.sync_copy(acc.at[pl.ds(r0, _NROWS_T)], out_hbm.at[c, pl.ds(r0, _NROWS_T)])

    f = pl.kernel(
        body,
        out_type=jax.ShapeDtypeStruct((_NC, _NP, _HALF), _f32),
        mesh=_sc_mesh(),
        compiler_params=pltpu.CompilerParams(use_tc_tiling_on_sc=False),
        scratch_types=[pltpu.VMEM((_EIB, _ECH), _i32)] * 4
        + [pltpu.VMEM((_ECH, _HALF), _f32)] * _NBUF + [
            pltpu.VMEM_SHARED((_NP, _HALF), _f32),
        ] + [pltpu.SemaphoreType.DMA] * (2 * _NBUF + 2),
    )
    return f(hsplit, src2, dst3, zblk)


def _sc_degree(dst3b, ones_blk, zblk):
    def body(ones_hbm, dst_hbm, z_hbm, out_hbm, dstv, rows, acc, ssem):
        c = lax.axis_index("c")
        s = lax.axis_index("s")
        r0 = s * _NROWS_T
        pltpu.sync_copy(z_hbm, acc.at[pl.ds(r0, _NROWS_T)])
        pltpu.sync_copy(ones_hbm, rows)
        plsc.subcore_barrier()

        def blk(j2, carry):
            pltpu.sync_copy(dst_hbm.at[c, s, pl.ds(j2 * _EIB, _EIB)], dstv)
            for j in range(_EIB):
                pltpu.async_copy(rows, acc.at[dstv.at[j]], ssem, add=True)
            for j in range(_EIB):
                pltpu.make_async_copy(rows, acc.at[dstv.at[j]], ssem).wait()
            return carry

        lax.fori_loop(0, _ENCH // _EIB, blk, 0)
        plsc.subcore_barrier()
        pltpu.sync_copy(acc.at[pl.ds(r0, _NROWS_T)], out_hbm.at[c, pl.ds(r0, _NROWS_T)])

    f = pl.kernel(
        body,
        out_type=jax.ShapeDtypeStruct((_NC, _NP, _HALF), _f32),
        mesh=_sc_mesh(),
        compiler_params=pltpu.CompilerParams(use_tc_tiling_on_sc=False),
        scratch_types=[
            pltpu.VMEM((_EIB, _ECH), _i32),
            pltpu.VMEM((_ECH, _HALF), _f32),
            pltpu.VMEM_SHARED((_NP, _HALF), _f32),
            pltpu.SemaphoreType.DMA,
        ],
    )
    return f(ones_blk, dst3b, zblk)


_PB = _NROWS_T


def _tc_proj(rows_pad, w_t, b):
    def body(x_ref, w_ref, b_ref, o_ref):
        x = jnp.dot(x_ref[...], w_ref[...], preferred_element_type=_f32) + b_ref[...]
        o_ref[0] = x[:, :_HALF]
        o_ref[1] = x[:, _HALF:]

    return pl.pallas_call(
        body,
        grid=(_NP // _PB,),
        in_specs=[
            pl.BlockSpec((_PB, rows_pad.shape[1]), lambda b: (b, 0)),
            pl.BlockSpec(w_t.shape, lambda b: (0, 0)),
            pl.BlockSpec((1, _HID), lambda b: (0, 0)),
        ],
        out_specs=pl.BlockSpec((_NC, _PB, _HALF), lambda b: (0, b, 0)),
        out_shape=jax.ShapeDtypeStruct((_NC, _NP, _HALF), _f32),
    )(rows_pad, w_t, b.reshape(1, _HID))


def _tc_gru(s2, h2, deg, wm_t, bm, wih_t, bih, whh_t, bhh):
    def body(s_ref, h_ref, d_ref, wm_ref, bm_ref, wih_ref, bih_ref,
             whh_ref, bhh_ref, o_ref):
        sv = jnp.concatenate([s_ref[0], s_ref[1]], axis=-1)
        hv = jnp.concatenate([h_ref[0], h_ref[1]], axis=-1)
        x = (jnp.dot(sv, wm_ref[...], preferred_element_type=_f32)
             + d_ref[...] * bm_ref[...])
        gi = jnp.dot(x, wih_ref[...], preferred_element_type=_f32) + bih_ref[...]
        gh = jnp.dot(hv, whh_ref[...], preferred_element_type=_f32) + bhh_ref[...]
        r = jax.nn.sigmoid(gi[:, :_HID] + gh[:, :_HID])
        z = jax.nn.sigmoid(gi[:, _HID:2 * _HID] + gh[:, _HID:2 * _HID])
        n = jnp.tanh(gi[:, 2 * _HID:] + r * gh[:, 2 * _HID:])
        hn = (1.0 - z) * n + z * hv
        o_ref[0] = hn[:, :_HALF]
        o_ref[1] = hn[:, _HALF:]

    full = lambda a: pl.BlockSpec(a.shape, lambda b: (0, 0))
    args = (s2, h2, deg, wm_t, bm, wih_t, bih, whh_t, bhh)
    split_spec = pl.BlockSpec((_NC, _PB, _HALF), lambda b: (0, b, 0))
    in_specs = [split_spec, split_spec, pl.BlockSpec((_PB, 1), lambda b: (b, 0))]
    in_specs += [full(a) for a in args[3:]]
    return pl.pallas_call(
        body,
        grid=(_NP // _PB,),
        in_specs=in_specs,
        out_specs=split_spec,
        out_shape=jax.ShapeDtypeStruct((_NC, _NP, _HALF), _f32),
    )(*args)


def _tc_classifier(pg0, pg1, y, w1_t, b1, w2_t, b2):
    p = y.shape[0]

    def body(a_ref, b_ref, c_ref, d_ref, y_ref, w1_ref, b1_ref, w2_ref, b2_ref,
             lg_ref, ls_ref):
        h = jnp.concatenate([a_ref[...], b_ref[...], c_ref[...], d_ref[...]],
                            axis=-1)
        h1 = jnp.dot(h, w1_ref[...], preferred_element_type=_f32) + b1_ref[...]
        h1 = jnp.maximum(h1, 0.0)
        u = jnp.dot(h1, w2_ref[...], preferred_element_type=_f32) + b2_ref[...]
        prob = jax.nn.sigmoid(u)
        lg_ref[...] = prob
        yv = y_ref[...]
        ll = (yv * jnp.maximum(jnp.log(prob), -100.0)
              + (1.0 - yv) * jnp.maximum(jnp.log(1.0 - prob), -100.0))
        ls_ref[...] = jnp.broadcast_to(-jnp.mean(ll), (1, 1))

    half_block = lambda i: pl.BlockSpec((p, _HALF), lambda b: (i, 0))
    full = lambda a: pl.BlockSpec(a.shape, lambda b: (0, 0))
    return pl.pallas_call(
        body,
        grid=(1,),
        in_specs=[half_block(0), half_block(1), half_block(0), half_block(1),
                  full(y), full(w1_t), full(b1), full(w2_t), full(b2)],
        out_specs=[pl.BlockSpec((p, 1), lambda b: (0, 0)),
                   pl.BlockSpec((1, 1), lambda b: (0, 0))],
        out_shape=[jax.ShapeDtypeStruct((p, 1), _f32),
                   jax.ShapeDtypeStruct((1, 1), _f32)],
    )(pg0, pg0, pg1, pg1, y, w1_t, b1, w2_t, b2)


def kernel(emb_table, proj_W, proj_b, msg_W_0, msg_b_0, gru_Wih_0, gru_Whh_0,
           gru_bih_0, gru_bhh_0, msg_W_1, msg_b_1, gru_Wih_1, gru_Whh_1,
           gru_bih_1, gru_bhh_1, cla1_W, cla1_b, cla2_W, cla2_b,
           emb_ind_0, emb_ind_1, adj_0, adj_1, prop_ind_0, prop_ind_1, labels):
    zblk = jnp.zeros((_NROWS_T, _HALF), _f32)
    ones_blk = jnp.ones((_ECH, _HALF), _f32)

    layers = []
    for msg_W, msg_b, Wih, Whh, bih, bhh in (
            (msg_W_0, msg_b_0, gru_Wih_0, gru_Whh_0, gru_bih_0, gru_bhh_0),
            (msg_W_1, msg_b_1, gru_Wih_1, gru_Whh_1, gru_bih_1, gru_bhh_1)):
        layers.append((msg_W.T, msg_b.reshape(1, _HID), Wih.T,
                       bih.reshape(1, 3 * _HID), Whh.T, bhh.reshape(1, 3 * _HID)))

    emb_nch, emb_ch = 14, 112
    emb_pad = _NW * emb_nch * emb_ch - _N

    hs, src2s, dst3s = [], [], []
    for emb_ind, adj in ((emb_ind_0, adj_0), (emb_ind_1, adj_1)):
        ei = jnp.concatenate([emb_ind.astype(_i32), jnp.zeros((emb_pad,), _i32)])
        rows = _sc_gather(emb_table, ei.reshape(_NW, emb_nch, emb_ch), 128)
        h2 = _tc_proj(rows, proj_W.T, proj_b)
        src = adj[:, 0].astype(_i32).reshape(_NS, _ENCH, _ECH)
        dst3 = adj[:, 1].astype(_i32).reshape(_NS, _ENCH, _ECH)
        src2 = jnp.stack([src, src + _NP])
        hs.append(h2)
        src2s.append(src2)
        dst3s.append(dst3)
    degb = _sc_degree(jnp.stack(dst3s), ones_blk, zblk)
    degs = [degb[g, :, :1] for g in range(2)]

    for wm_t, bm, wih_t, bih, whh_t, bhh in layers:
        for _ in range(3):
            s2 = [_sc_segsum(hs[g].reshape(_NC * _NP, _HALF), src2s[g], dst3s[g],
                             zblk) for g in range(2)]
            hs = [_tc_gru(s2[g], hs[g], degs[g], wm_t, bm, wih_t, bih,
                          whh_t, bhh) for g in range(2)]

    pgs = []
    for g, prop_ind in enumerate((prop_ind_0, prop_ind_1)):
        pi = prop_ind.astype(_i32)
        pidx = jnp.concatenate([pi, pi + _NP]).reshape(_NW, 2, 128)
        pgs.append(_sc_gather(hs[g].reshape(_NC * _NP, _HALF), pidx, _HALF))

    y = labels.astype(_f32).reshape(-1, 1)
    logits, loss = _tc_classifier(pgs[0], pgs[1], y, cla1_W.T,
                                  cla1_b.reshape(1, _HID), cla2_W.T,
                                  cla2_b.reshape(1, 1))
    return logits, loss.reshape(())


# speedup vs baseline: 1.0660x; 1.0043x over previous
import functools

import jax
import jax.numpy as jnp
from jax import lax
from jax.experimental import pallas as pl
from jax.experimental.pallas import tpu as pltpu
from jax.experimental.pallas import tpu_sc as plsc

_N = 50000
_NP = 50176
_E = 800000
_HID = 64
_HALF = 32
_NC = 2
_NS = 16
_NW = _NC * _NS
_ECH = 125
_ENCH = _E // (_NS * _ECH)
_EIB = 20
_NBUF = 5
_NROWS_T = _NP // _NS

_f32 = jnp.float32
_i32 = jnp.int32


def _sc_mesh():
    return plsc.VectorSubcoreMesh(core_axis_name="c", subcore_axis_name="s",
                                  num_cores=_NC, num_subcores=_NS)


def _sc_gather(table, idx3, d):
    nw, nch, ch = idx3.shape

    def body(table_hbm, idx_hbm, out_hbm, idxv, rows0, rows1, rows2,
             gs0, gs1, gs2, os0, os1, os2):
        c = lax.axis_index("c")
        s = lax.axis_index("s")
        w = c * _NS + s
        rows = (rows0, rows1, rows2)
        gsem = (gs0, gs1, gs2)
        osem = (os0, os1, os2)
        pltpu.sync_copy(idx_hbm.at[w], idxv)
        for k in range(min(3, nch)):
            pltpu.async_copy(table_hbm.at[idxv.at[k]], rows[k], gsem[k])
        for k in range(nch):
            b = k % 3
            pltpu.make_async_copy(table_hbm.at[idxv.at[k]], rows[b], gsem[b]).wait()
            pltpu.async_copy(rows[b], out_hbm.at[w, pl.ds(k * ch, ch)], osem[b])
            if k >= 1:
                bp = (k - 1) % 3
                pltpu.make_async_copy(rows[bp], out_hbm.at[w, pl.ds(0, ch)], osem[bp]).wait()
                kn = k + 2
                if kn < nch:
                    pltpu.async_copy(table_hbm.at[idxv.at[kn]], rows[bp], gsem[bp])
        bl = (nch - 1) % 3
        pltpu.make_async_copy(rows[bl], out_hbm.at[w, pl.ds(0, ch)], osem[bl]).wait()

    f = pl.kernel(
        body,
        out_type=jax.ShapeDtypeStruct((nw, nch * ch, d), _f32),
        mesh=_sc_mesh(),
        compiler_params=pltpu.CompilerParams(use_tc_tiling_on_sc=False),
        scratch_types=[pltpu.VMEM((nch, ch), _i32)]
        + [pltpu.VMEM((ch, d), _f32)] * 3
        + [pltpu.SemaphoreType.DMA] * 6,
    )
    return f(table, idx3).reshape(nw * nch * ch, d)


def _sc_segsum(hsplit, src2, dst3, zblk):
    def body(h_hbm, src_hbm, dst_hbm, z_hbm, out_hbm, srcv0, dstv0, srcv1, dstv1,
             rows0, rows1, rows2, rows3, rows4, acc,
             gs0, gs1, gs2, gs3, gs4, ss0, ss1, ss2, ss3, ss4, is0, is1):
        c = lax.axis_index("c")
        s = lax.axis_index("s")
        r0 = s * _NROWS_T
        rows = (rows0, rows1, rows2, rows3, rows4)
        gsem = (gs0, gs1, gs2, gs3, gs4)
        ssem = (ss0, ss1, ss2, ss3, ss4)
        slots = ((srcv0, dstv0, is0), (srcv1, dstv1, is1))
        nblk = _ENCH // _EIB
        pltpu.async_copy(z_hbm, acc.at[pl.ds(r0, _NROWS_T)], ss0)
        pltpu.async_copy(src_hbm.at[c, s, pl.ds(0, _EIB)], srcv0, is0)
        pltpu.async_copy(dst_hbm.at[s, pl.ds(0, _EIB)], dstv0, is0)
        pltpu.make_async_copy(z_hbm, acc.at[pl.ds(r0, _NROWS_T)], ss0).wait()
        plsc.subcore_barrier()

        def run_block(j, sv, dv, isem, sv_n, dv_n, isem_n):
            pltpu.make_async_copy(src_hbm.at[c, s, pl.ds(0, _EIB)], sv, isem).wait()
            pltpu.make_async_copy(dst_hbm.at[s, pl.ds(0, _EIB)], dv, isem).wait()
            jn = jnp.minimum(j + 1, nblk - 1) * _EIB
            pltpu.async_copy(src_hbm.at[c, s, pl.ds(jn, _EIB)], sv_n, isem_n)
            pltpu.async_copy(dst_hbm.at[s, pl.ds(jn, _EIB)], dv_n, isem_n)
            for k in range(_NBUF):
                pltpu.async_copy(h_hbm.at[sv.at[k]], rows[k], gsem[k])
            for k in range(_EIB):
                b = k % _NBUF
                pltpu.make_async_copy(h_hbm.at[sv.at[k]], rows[b], gsem[b]).wait()
                pltpu.async_copy(rows[b], acc.at[dv.at[k]], ssem[b], add=True)
                if k >= 1:
                    bp = (k - 1) % _NBUF
                    pltpu.make_async_copy(rows[bp], acc.at[dv.at[k]], ssem[bp]).wait()
                    kn = k + _NBUF - 1
                    if kn < _EIB:
                        pltpu.async_copy(h_hbm.at[sv.at[kn]], rows[bp], gsem[bp])
            bl = (_EIB - 1) % _NBUF
            pltpu.make_async_copy(rows[bl], acc.at[dv.at[0]], ssem[bl]).wait()

        def sblk(i, carry):
            run_block(2 * i, *slots[0], *slots[1])
            run_block(2 * i + 1, *slots[1], *slots[0])
            return carry

        lax.fori_loop(0, nblk // 2, sblk, 0)
        pltpu.make_async_copy(src_hbm.at[c, s, pl.ds(0, _EIB)], srcv0, is0).wait()
        pltpu.make_async_copy(dst_hbm.at[s, pl.ds(0, _EIB)], dstv0, is0).wait()
        plsc.subcore_barrier()
        pltpu.sync_copy(acc.at[pl.ds(r0, _NROWS_T)], out_hbm.at[c, pl.ds(r0, _NROWS_T)])

    f = pl.kernel(
        body,
        out_type=jax.ShapeDtypeStruct((_NC, _NP, _HALF), _f32),
        mesh=_sc_mesh(),
        compiler_params=pltpu.CompilerParams(use_tc_tiling_on_sc=False),
        scratch_types=[pltpu.VMEM((_EIB, _ECH), _i32)] * 4
        + [pltpu.VMEM((_ECH, _HALF), _f32)] * _NBUF + [
            pltpu.VMEM_SHARED((_NP, _HALF), _f32),
        ] + [pltpu.SemaphoreType.DMA] * (2 * _NBUF + 2),
    )
    return f(hsplit, src2, dst3, zblk)


def _sc_degree(dst3b, ones_blk, zblk):
    def body(ones_hbm, dst_hbm, z_hbm, out_hbm, dstv, rows, acc, ssem):
        c = lax.axis_index("c")
        s = lax.axis_index("s")
        r0 = s * _NROWS_T
        pltpu.sync_copy(z_hbm, acc.at[pl.ds(r0, _NROWS_T)])
        pltpu.sync_copy(ones_hbm, rows)
        plsc.subcore_barrier()

        def blk(j2, carry):
            pltpu.sync_copy(dst_hbm.at[c, s, pl.ds(j2 * _EIB, _EIB)], dstv)
            for j in range(_EIB):
                pltpu.async_copy(rows, acc.at[dstv.at[j]], ssem, add=True)
            for j in range(_EIB):
                pltpu.make_async_copy(rows, acc.at[dstv.at[j]], ssem).wait()
            return carry

        lax.fori_loop(0, _ENCH // _EIB, blk, 0)
        plsc.subcore_barrier()
        pltpu.sync_copy(acc.at[pl.ds(r0, _NROWS_T)], out_hbm.at[c, pl.ds(r0, _NROWS_T)])

    f = pl.kernel(
        body,
        out_type=jax.ShapeDtypeStruct((_NC, _NP, _HALF), _f32),
        mesh=_sc_mesh(),
        compiler_params=pltpu.CompilerParams(use_tc_tiling_on_sc=False),
        scratch_types=[
            pltpu.VMEM((_EIB, _ECH), _i32),
            pltpu.VMEM((_ECH, _HALF), _f32),
            pltpu.VMEM_SHARED((_NP, _HALF), _f32),
            pltpu.SemaphoreType.DMA,
        ],
    )
    return f(ones_blk, dst3b, zblk)


_PB = _NROWS_T


def _tc_proj(rows_merged, g, w_t, b):
    ch2 = _NP // _NW

    def body(x_ref, w_ref, b_ref, o_ref):
        x = jnp.dot(x_ref[...], w_ref[...], preferred_element_type=_f32) + b_ref[...]
        o_ref[0] = x[:, :_HALF]
        o_ref[1] = x[:, _HALF:]

    return pl.pallas_call(
        body,
        grid=(_NW,),
        in_specs=[
            pl.BlockSpec((ch2, rows_merged.shape[1]), lambda b: (2 * b + g, 0)),
            pl.BlockSpec(w_t.shape, lambda b: (0, 0)),
            pl.BlockSpec((1, _HID), lambda b: (0, 0)),
        ],
        out_specs=pl.BlockSpec((_NC, ch2, _HALF), lambda b: (0, b, 0)),
        out_shape=jax.ShapeDtypeStruct((_NC, _NP, _HALF), _f32),
    )(rows_merged, w_t, b.reshape(1, _HID))


def _tc_gru(s2, h2, deg, wm_t, bm, wih_t, bih, whh_t, bhh):
    def body(s_ref, h_ref, d_ref, wm_ref, bm_ref, wih_ref, bih_ref,
             whh_ref, bhh_ref, o_ref):
        sv = jnp.concatenate([s_ref[0], s_ref[1]], axis=-1)
        hv = jnp.concatenate([h_ref[0], h_ref[1]], axis=-1)
        x = (jnp.dot(sv, wm_ref[...], preferred_element_type=_f32)
             + d_ref[...] * bm_ref[...])
        gi = jnp.dot(x, wih_ref[...], preferred_element_type=_f32) + bih_ref[...]
        gh = jnp.dot(hv, whh_ref[...], preferred_element_type=_f32) + bhh_ref[...]
        r = jax.nn.sigmoid(gi[:, :_HID] + gh[:, :_HID])
        z = jax.nn.sigmoid(gi[:, _HID:2 * _HID] + gh[:, _HID:2 * _HID])
        n = jnp.tanh(gi[:, 2 * _HID:] + r * gh[:, 2 * _HID:])
        hn = (1.0 - z) * n + z * hv
        o_ref[0] = hn[:, :_HALF]
        o_ref[1] = hn[:, _HALF:]

    full = lambda a: pl.BlockSpec(a.shape, lambda b: (0, 0))
    args = (s2, h2, deg, wm_t, bm, wih_t, bih, whh_t, bhh)
    split_spec = pl.BlockSpec((_NC, _PB, _HALF), lambda b: (0, b, 0))
    in_specs = [split_spec, split_spec, pl.BlockSpec((_PB, 1), lambda b: (b, 0))]
    in_specs += [full(a) for a in args[3:]]
    return pl.pallas_call(
        body,
        grid=(_NP // _PB,),
        in_specs=in_specs,
        out_specs=split_spec,
        out_shape=jax.ShapeDtypeStruct((_NC, _NP, _HALF), _f32),
    )(*args)


def _tc_classifier(pg0, pg1, y, w1_t, b1, w2_t, b2):
    p = y.shape[0]

    def body(a_ref, b_ref, c_ref, d_ref, y_ref, w1_ref, b1_ref, w2_ref, b2_ref,
             lg_ref, ls_ref):
        h = jnp.concatenate([a_ref[...], b_ref[...], c_ref[...], d_ref[...]],
                            axis=-1)
        h1 = jnp.dot(h, w1_ref[...], preferred_element_type=_f32) + b1_ref[...]
        h1 = jnp.maximum(h1, 0.0)
        u = jnp.dot(h1, w2_ref[...], preferred_element_type=_f32) + b2_ref[...]
        prob = jax.nn.sigmoid(u)
        lg_ref[...] = prob
        yv = y_ref[...]
        ll = (yv * jnp.maximum(jnp.log(prob), -100.0)
              + (1.0 - yv) * jnp.maximum(jnp.log(1.0 - prob), -100.0))
        ls_ref[...] = jnp.broadcast_to(-jnp.mean(ll), (1, 1))

    half_block = lambda i: pl.BlockSpec((p, _HALF), lambda b: (i, 0))
    full = lambda a: pl.BlockSpec(a.shape, lambda b: (0, 0))
    return pl.pallas_call(
        body,
        grid=(1,),
        in_specs=[half_block(0), half_block(1), half_block(0), half_block(1),
                  full(y), full(w1_t), full(b1), full(w2_t), full(b2)],
        out_specs=[pl.BlockSpec((p, 1), lambda b: (0, 0)),
                   pl.BlockSpec((1, 1), lambda b: (0, 0))],
        out_shape=[jax.ShapeDtypeStruct((p, 1), _f32),
                   jax.ShapeDtypeStruct((1, 1), _f32)],
    )(pg0, pg0, pg1, pg1, y, w1_t, b1, w2_t, b2)


def kernel(emb_table, proj_W, proj_b, msg_W_0, msg_b_0, gru_Wih_0, gru_Whh_0,
           gru_bih_0, gru_bhh_0, msg_W_1, msg_b_1, gru_Wih_1, gru_Whh_1,
           gru_bih_1, gru_bhh_1, cla1_W, cla1_b, cla2_W, cla2_b,
           emb_ind_0, emb_ind_1, adj_0, adj_1, prop_ind_0, prop_ind_1, labels):
    zblk = jnp.zeros((_NROWS_T, _HALF), _f32)
    ones_blk = jnp.ones((_ECH, _HALF), _f32)

    layers = []
    for msg_W, msg_b, Wih, Whh, bih, bhh in (
            (msg_W_0, msg_b_0, gru_Wih_0, gru_Whh_0, gru_bih_0, gru_bhh_0),
            (msg_W_1, msg_b_1, gru_Wih_1, gru_Whh_1, gru_bih_1, gru_bhh_1)):
        layers.append((msg_W.T, msg_b.reshape(1, _HID), Wih.T,
                       bih.reshape(1, 3 * _HID), Whh.T, bhh.reshape(1, 3 * _HID)))

    emb_nch, emb_ch = 14, 112
    emb_pad = _NP - _N

    eis = []
    for emb_ind in (emb_ind_0, emb_ind_1):
        ei = jnp.concatenate([emb_ind.astype(_i32), jnp.zeros((emb_pad,), _i32)])
        eis.append(ei.reshape(_NW, emb_nch, emb_ch))
    ei_m = jnp.concatenate(eis, axis=1)
    rows_m = _sc_gather(emb_table, ei_m, 128)

    hs, src2s, dst3s = [], [], []
    for g, adj in enumerate((adj_0, adj_1)):
        h2 = _tc_proj(rows_m, g, proj_W.T, proj_b)
        src = adj[:, 0].astype(_i32).reshape(_NS, _ENCH, _ECH)
        dst3 = adj[:, 1].astype(_i32).reshape(_NS, _ENCH, _ECH)
        src2 = jnp.stack([src, src + _NP])
        hs.append(h2)
        src2s.append(src2)
        dst3s.append(dst3)
    degb = _sc_degree(jnp.stack(dst3s), ones_blk, zblk)
    degs = [degb[g, :, :1] for g in range(2)]

    for wm_t, bm, wih_t, bih, whh_t, bhh in layers:
        for _ in range(3):
            s2 = [_sc_segsum(hs[g].reshape(_NC * _NP, _HALF), src2s[g], dst3s[g],
                             zblk) for g in range(2)]
            hs = [_tc_gru(s2[g], hs[g], degs[g], wm_t, bm, wih_t, bih,
                          whh_t, bhh) for g in range(2)]

    pgs = []
    for g, prop_ind in enumerate((prop_ind_0, prop_ind_1)):
        pi = prop_ind.astype(_i32)
        pidx = jnp.concatenate([pi, pi + _NP]).reshape(_NW, 2, 128)
        pgs.append(_sc_gather(hs[g].reshape(_NC * _NP, _HALF), pidx, _HALF))

    y = labels.astype(_f32).reshape(-1, 1)
    logits, loss = _tc_classifier(pgs[0], pgs[1], y, cla1_W.T,
                                  cla1_b.reshape(1, _HID), cla2_W.T,
                                  cla2_b.reshape(1, 1))
    return logits, loss.reshape(())
